# baseline (speedup 1.0000x reference)
import functools

import jax
import jax.numpy as jnp
from jax import lax
from jax.experimental import pallas as pl
from jax.experimental.pallas import tpu as pltpu
from jax.experimental.pallas import tpu_sc as plsc

VOCAB = 100000
DIM = 64
PAD = 1
B = 1024
M = 50
T = 6
NC, NS, L = 2, 16, 16
NW = NC * NS
BATW = B // NW
NCH = M // 2
RR = 2 * T * BATW
JP = 32
BF = jnp.bfloat16
LB = 2 * L


def _sc_pair_gather(tp_in, idx_arr):
    mesh = plsc.VectorSubcoreMesh(
        core_axis_name="c", subcore_axis_name="s",
        num_cores=NC, num_subcores=NS)
    out_t = tuple(jax.ShapeDtypeStruct((B, JP, 2 * DIM), BF)
                  for _ in range(2))
    RRP = 2 * T * BATW
    scratch = [
        pltpu.VMEM((T, M, BATW), jnp.int32),
        pltpu.VMEM((RRP, 2 * DIM), BF),
        pltpu.VMEM((RRP, 2 * DIM), BF),
        pltpu.VMEM((2, BATW, 2 * DIM), BF),
        pltpu.VMEM((2, BATW, 2 * DIM), BF),
        pltpu.SemaphoreType.DMA,
        pltpu.SemaphoreType.DMA,
        pltpu.SemaphoreType.DMA,
        pltpu.SemaphoreType.DMA,
    ]

    @functools.partial(pl.kernel, mesh=mesh, out_type=out_t,
                       scratch_types=scratch,
                       compiler_params=pltpu.CompilerParams(
                           use_tc_tiling_on_sc=False))
    def k(tp, idx_hbm, oa, ob,
          idx_v, rows0, rows1, acc0, acc1, semg0, semg1, semw0, semw1):
        cid = lax.axis_index("c")
        sid = lax.axis_index("s")
        wid = sid * NC + cid
        pltpu.sync_copy(
            idx_hbm.at[pl.ds(0, T), pl.ds(0, M),
                       pl.ds(wid * BATW, BATW)], idx_v)

        def fire(j, buf, sem):
            for par in range(2):
                for t in range(T):
                    pltpu.async_copy(
                        tp.at[idx_v.at[t, 2 * j + par]],
                        buf.at[pl.ds((par * T + t) * BATW, BATW)], sem)

        def drain_g(buf, sem):
            pltpu.make_async_copy(tp.at[pl.ds(0, RRP)], buf, sem).wait()

        def pool(buf, acc):
            @pl.loop(0, BATW)
            def _pool(bb):
                for par in range(2):
                    base = par * T * BATW + bb
                    for d in range(2 * DIM // LB):
                        sl = pl.ds(d * LB, LB)
                        v01 = buf[base, sl] + buf[base + BATW, sl]
                        v23 = (buf[base + 2 * BATW, sl]
                               + buf[base + 3 * BATW, sl])
                        v45 = (buf[base + 4 * BATW, sl]
                               + buf[base + 5 * BATW, sl])
                        acc[par, bb, sl] = (v01 + v23) + v45

        def wb(j, acc, semw):
            for par in range(2):
                pltpu.async_copy(
                    acc.at[par, pl.ds(0, BATW), pl.ds(0, DIM)],
                    oa.at[pl.ds(wid * BATW, BATW), j,
                          pl.ds(par * DIM, DIM)], semw)
                pltpu.async_copy(
                    acc.at[par, pl.ds(0, BATW), pl.ds(DIM, DIM)],
                    ob.at[pl.ds(wid * BATW, BATW), j,
                          pl.ds(par * DIM, DIM)], semw)

        def drain_w(acc, semw):
            for par in range(2):
                for _ in range(2):
                    pltpu.make_async_copy(
                        oa.at[pl.ds(0, BATW), 0, pl.ds(0, DIM)],
                        acc.at[par, pl.ds(0, BATW), pl.ds(0, DIM)],
                        semw).wait()

        fire(0, rows0, semg0)

        @pl.loop(0, NCH - 1, step=2)
        def _pairs(j):
            fire(j + 1, rows1, semg1)
            drain_g(rows0, semg0)

            @pl.when(j >= 2)
            def _():
                drain_w(acc0, semw0)

            pool(rows0, acc0)
            wb(j, acc0, semw0)

            @pl.when(j + 2 < NCH)
            def _():
                fire(j + 2, rows0, semg0)

            drain_g(rows1, semg1)

            @pl.when(j >= 2)
            def _():
                drain_w(acc1, semw1)

            pool(rows1, acc1)
            wb(j + 1, acc1, semw1)

        drain_g(rows0, semg0)
        drain_w(acc0, semw0)
        pool(rows0, acc0)
        wb(NCH - 1, acc0, semw0)
        drain_w(acc0, semw0)
        drain_w(acc1, semw1)

        @pl.loop(0, BATW)
        def _z(bb):
            for d in range(2 * DIM // LB):
                rows0[bb, pl.ds(d * LB, LB)] = jnp.zeros((LB,), BF)

        for i in range(JP - NCH):
            for o in (oa, ob):
                pltpu.async_copy(
                    rows0.at[pl.ds(0, BATW)],
                    o.at[pl.ds(wid * BATW, BATW), NCH + i,
                         pl.ds(0, 2 * DIM)], semg0)
        for i in range(JP - NCH):
            for o in (oa, ob):
                pltpu.make_async_copy(
                    o.at[pl.ds(0, BATW), 0, pl.ds(0, 2 * DIM)],
                    rows0.at[pl.ds(0, BATW)], semg0).wait()

    return k(tp_in, idx_arr)


def _sc_pooled_gather(tbl_in, idx_arr):
    mesh = plsc.VectorSubcoreMesh(
        core_axis_name="c", subcore_axis_name="s",
        num_cores=NC, num_subcores=NS)
    out_t = jax.ShapeDtypeStruct((B, JP, 2 * DIM), BF)
    scratch = [
        pltpu.VMEM((T, M, BATW), jnp.int32),
        pltpu.VMEM((RR, DIM), BF),
        pltpu.VMEM((RR, DIM), BF),
        pltpu.VMEM((2, BATW, DIM), BF),
        pltpu.VMEM((2, BATW, DIM), BF),
        pltpu.SemaphoreType.DMA,
        pltpu.SemaphoreType.DMA,
        pltpu.SemaphoreType.DMA,
        pltpu.SemaphoreType.DMA,
    ]

    @functools.partial(pl.kernel, mesh=mesh, out_type=out_t,
                       scratch_types=scratch,
                       compiler_params=pltpu.CompilerParams(
                           use_tc_tiling_on_sc=False))
    def k(tbl, idx_hbm, out,
          idx_v, rows0, rows1, accum0, accum1, semg0, semg1, semw0, semw1):
        cid = lax.axis_index("c")
        sid = lax.axis_index("s")
        wid = sid * NC + cid
        pltpu.sync_copy(
            idx_hbm.at[pl.ds(0, T), pl.ds(0, M),
                       pl.ds(wid * BATW, BATW)], idx_v)

        def fire(c, buf, sem):
            for par in range(2):
                for t in range(T):
                    pltpu.async_copy(
                        tbl.at[idx_v.at[t, 2 * c + par]],
                        buf.at[pl.ds((par * T + t) * BATW, BATW)],
                        sem)

        def drain_g(buf, sem):
            pltpu.make_async_copy(tbl.at[pl.ds(0, RR)], buf, sem).wait()

        def pool(buf, acc):
            @pl.loop(0, BATW)
            def _pool(bb):
                for par in range(2):
                    base = par * T * BATW + bb
                    for d in range(DIM // LB):
                        sl = pl.ds(d * LB, LB)
                        v01 = buf[base, sl] + buf[base + BATW, sl]
                        v23 = (buf[base + 2 * BATW, sl]
                               + buf[base + 3 * BATW, sl])
                        v45 = (buf[base + 4 * BATW, sl]
                               + buf[base + 5 * BATW, sl])
                        acc[par, bb, sl] = (v01 + v23) + v45

        def wb(c, acc, semw):
            for par in range(2):
                pltpu.async_copy(
                    acc.at[par],
                    out.at[pl.ds(wid * BATW, BATW), c,
                           pl.ds(par * DIM, DIM)], semw)

        def drain_w(acc, semw):
            for par in range(2):
                pltpu.make_async_copy(
                    out.at[pl.ds(0, BATW), 0, pl.ds(0, DIM)],
                    acc.at[par], semw).wait()

        fire(0, rows0, semg0)

        @pl.loop(0, NCH - 1, step=2)
        def _chunks(c):
            fire(c + 1, rows1, semg1)
            drain_g(rows0, semg0)

            @pl.when(c >= 2)
            def _():
                drain_w(accum0, semw0)

            pool(rows0, accum0)
            wb(c, accum0, semw0)

            @pl.when(c + 2 < NCH)
            def _():
                fire(c + 2, rows0, semg0)

            drain_g(rows1, semg1)

            @pl.when(c >= 2)
            def _():
                drain_w(accum1, semw1)

            pool(rows1, accum1)
            wb(c + 1, accum1, semw1)

        drain_g(rows0, semg0)
        drain_w(accum0, semw0)
        pool(rows0, accum0)
        wb(NCH - 1, accum0, semw0)
        drain_w(accum0, semw0)
        drain_w(accum1, semw1)

        @pl.loop(0, BATW)
        def _z(bb):
            for d in range(DIM // LB):
                rows0[bb, pl.ds(d * LB, LB)] = jnp.zeros((LB,), BF)

        for i in range(JP - NCH):
            for par in range(2):
                pltpu.async_copy(
                    rows0.at[pl.ds(0, BATW)],
                    out.at[pl.ds(wid * BATW, BATW), NCH + i,
                           pl.ds(par * DIM, DIM)], semg0)
        for i in range(2 * (JP - NCH)):
            pltpu.make_async_copy(
                out.at[pl.ds(0, BATW), 0, pl.ds(0, DIM)],
                rows0.at[pl.ds(0, BATW)], semg0).wait()

    return k(tbl_in, idx_arr)


def _tc_hops(s1, s2, s3):
    Bb = 256

    def body(s1_ref, s2_ref, s3_ref, o_ref):
        m1 = lax.broadcasted_iota(jnp.int32, (Bb, JP, 1), 1) < 25
        lane_e = lax.broadcasted_iota(jnp.int32, (Bb, JP, 2 * DIM), 2) < DIM
        s1 = s1_ref[...].astype(jnp.float32)
        s2 = s2_ref[...].astype(jnp.float32)
        s3 = s3_ref[...].astype(jnp.float32)
        su1 = jnp.sum(s1, axis=1, keepdims=True)
        u = (su1[:, :, 0:DIM] + su1[:, :, DIM:2 * DIM]) / float(M)
        neg = jnp.float32(-1e30)
        for sa, sc in ((s1, s2), (s2, s3)):
            u2 = jnp.concatenate([u, u], axis=2)
            prod = sa * u2
            ae = jnp.sum(prod[:, :, 0:DIM], axis=2, keepdims=True)
            ao = jnp.sum(prod[:, :, DIM:2 * DIM], axis=2, keepdims=True)
            ae = jnp.where(m1, ae, neg)
            ao = jnp.where(m1, ao, neg)
            mx = jnp.maximum(jnp.max(ae, axis=1, keepdims=True),
                             jnp.max(ao, axis=1, keepdims=True))
            ee = jnp.exp(ae - mx)
            eo = jnp.exp(ao - mx)
            z = jnp.sum(ee, axis=1, keepdims=True) + jnp.sum(
                eo, axis=1, keepdims=True)
            p = jnp.where(lane_e, ee, eo) / z
            o2 = jnp.sum(sc * p, axis=1, keepdims=True)
            u = u + o2[:, :, 0:DIM] + o2[:, :, DIM:2 * DIM]
        o_ref[...] = u

    out = pl.pallas_call(
        body,
        grid=(B // Bb,),
        in_specs=[
            pl.BlockSpec((Bb, JP, 2 * DIM), lambda i: (i, 0, 0)),
            pl.BlockSpec((Bb, JP, 2 * DIM), lambda i: (i, 0, 0)),
            pl.BlockSpec((Bb, JP, 2 * DIM), lambda i: (i, 0, 0)),
        ],
        out_specs=pl.BlockSpec((Bb, 1, DIM), lambda i: (i, 0, 0)),
        out_shape=jax.ShapeDtypeStruct((B, 1, DIM), jnp.float32),
    )(s1, s2, s3)
    return out.reshape(B, DIM)


def kernel(story, C_0, C_1, C_2, C_3):
    tp12 = jnp.concatenate(
        [C_1.at[PAD].set(0.0).astype(BF),
         C_2.at[PAD].set(0.0).astype(BF)], axis=1)
    t3 = jnp.pad(C_3.astype(BF), ((0, 0), (0, DIM))).reshape(2 * VOCAB, DIM)
    js = jnp.transpose(story, (2, 0, 1))
    idx3 = jnp.where(js == PAD, 1, js * 2)
    S1, S2 = _sc_pair_gather(tp12, js)
    S3 = _sc_pooled_gather(t3, idx3)
    return _tc_hops(S1, S2, S3)


# speedup vs baseline: 1.6701x; 1.6701x over previous
import functools

import jax
import jax.numpy as jnp
from jax import lax
from jax.experimental import pallas as pl
from jax.experimental.pallas import tpu as pltpu
from jax.experimental.pallas import tpu_sc as plsc

VOCAB = 100000
DIM = 64
PAD = 1
B = 1024
M = 50
T = 6
NC, NS, L = 2, 16, 16
NW = NC * NS
BATW = B // NW
NCH = M // 2
RR = 2 * T * BATW
JP = 32


def _sc_pair_gather(tp_in, idx_arr):
    mesh = plsc.VectorSubcoreMesh(
        core_axis_name="c", subcore_axis_name="s",
        num_cores=NC, num_subcores=NS)
    out_t = tuple(jax.ShapeDtypeStruct((B, JP, 2 * DIM), jnp.float32)
                  for _ in range(2))
    RRP = 2 * T * BATW
    scratch = [
        pltpu.VMEM((T, M, BATW), jnp.int32),
        pltpu.VMEM((RRP, 2 * DIM), jnp.float32),
        pltpu.VMEM((RRP, 2 * DIM), jnp.float32),
        pltpu.VMEM((2, BATW, 2 * DIM), jnp.float32),
        pltpu.VMEM((2, BATW, 2 * DIM), jnp.float32),
        pltpu.SemaphoreType.DMA,
        pltpu.SemaphoreType.DMA,
        pltpu.SemaphoreType.DMA,
        pltpu.SemaphoreType.DMA,
    ]

    @functools.partial(pl.kernel, mesh=mesh, out_type=out_t,
                       scratch_types=scratch,
                       compiler_params=pltpu.CompilerParams(
                           use_tc_tiling_on_sc=False))
    def k(tp, idx_hbm, oa, ob,
          idx_v, rows0, rows1, acc0, acc1, semg0, semg1, semw0, semw1):
        cid = lax.axis_index("c")
        sid = lax.axis_index("s")
        wid = sid * NC + cid
        pltpu.sync_copy(
            idx_hbm.at[pl.ds(0, T), pl.ds(0, M),
                       pl.ds(wid * BATW, BATW)], idx_v)

        def fire(j, buf, sem):
            for par in range(2):
                for t in range(T):
                    pltpu.async_copy(
                        tp.at[idx_v.at[t, 2 * j + par]],
                        buf.at[pl.ds((par * T + t) * BATW, BATW)], sem)

        def drain_g(buf, sem):
            pltpu.make_async_copy(tp.at[pl.ds(0, RRP)], buf, sem).wait()

        def pool(buf, acc):
            @pl.loop(0, BATW)
            def _pool(bb):
                for par in range(2):
                    base = par * T * BATW + bb
                    for d in range(2 * DIM // L):
                        sl = pl.ds(d * L, L)
                        v01 = buf[base, sl] + buf[base + BATW, sl]
                        v23 = (buf[base + 2 * BATW, sl]
                               + buf[base + 3 * BATW, sl])
                        v45 = (buf[base + 4 * BATW, sl]
                               + buf[base + 5 * BATW, sl])
                        acc[par, bb, sl] = (v01 + v23) + v45

        def wb(j, acc, semw):
            for par in range(2):
                pltpu.async_copy(
                    acc.at[par, pl.ds(0, BATW), pl.ds(0, DIM)],
                    oa.at[pl.ds(wid * BATW, BATW), j,
                          pl.ds(par * DIM, DIM)], semw)
                pltpu.async_copy(
                    acc.at[par, pl.ds(0, BATW), pl.ds(DIM, DIM)],
                    ob.at[pl.ds(wid * BATW, BATW), j,
                          pl.ds(par * DIM, DIM)], semw)

        def drain_w(acc, semw):
            for par in range(2):
                for _ in range(2):
                    pltpu.make_async_copy(
                        oa.at[pl.ds(0, BATW), 0, pl.ds(0, DIM)],
                        acc.at[par, pl.ds(0, BATW), pl.ds(0, DIM)],
                        semw).wait()

        fire(0, rows0, semg0)

        @pl.loop(0, NCH - 1, step=2)
        def _pairs(j):
            fire(j + 1, rows1, semg1)
            drain_g(rows0, semg0)

            @pl.when(j >= 2)
            def _():
                drain_w(acc0, semw0)

            pool(rows0, acc0)
            wb(j, acc0, semw0)

            @pl.when(j + 2 < NCH)
            def _():
                fire(j + 2, rows0, semg0)

            drain_g(rows1, semg1)

            @pl.when(j >= 2)
            def _():
                drain_w(acc1, semw1)

            pool(rows1, acc1)
            wb(j + 1, acc1, semw1)

        drain_g(rows0, semg0)
        drain_w(acc0, semw0)
        pool(rows0, acc0)
        wb(NCH - 1, acc0, semw0)
        drain_w(acc0, semw0)
        drain_w(acc1, semw1)

        @pl.loop(0, BATW)
        def _z(bb):
            for d in range(2 * DIM // L):
                rows0[bb, pl.ds(d * L, L)] = jnp.zeros((L,), jnp.float32)

        for i in range(JP - NCH):
            for o in (oa, ob):
                pltpu.async_copy(
                    rows0.at[pl.ds(0, BATW)],
                    o.at[pl.ds(wid * BATW, BATW), NCH + i,
                         pl.ds(0, 2 * DIM)], semg0)
        for i in range(JP - NCH):
            for o in (oa, ob):
                pltpu.make_async_copy(
                    o.at[pl.ds(0, BATW), 0, pl.ds(0, 2 * DIM)],
                    rows0.at[pl.ds(0, BATW)], semg0).wait()

    return k(tp_in, idx_arr)


def _sc_pooled_gather(tbl_in, idx_arr):
    mesh = plsc.VectorSubcoreMesh(
        core_axis_name="c", subcore_axis_name="s",
        num_cores=NC, num_subcores=NS)
    out_t = jax.ShapeDtypeStruct((B, JP, 2 * DIM), jnp.float32)
    scratch = [
        pltpu.VMEM((T, M, BATW), jnp.int32),
        pltpu.VMEM((RR, DIM), jnp.float32),
        pltpu.VMEM((RR, DIM), jnp.float32),
        pltpu.VMEM((2, BATW, DIM), jnp.float32),
        pltpu.VMEM((2, BATW, DIM), jnp.float32),
        pltpu.SemaphoreType.DMA,
        pltpu.SemaphoreType.DMA,
        pltpu.SemaphoreType.DMA,
        pltpu.SemaphoreType.DMA,
    ]

    @functools.partial(pl.kernel, mesh=mesh, out_type=out_t,
                       scratch_types=scratch,
                       compiler_params=pltpu.CompilerParams(
                           use_tc_tiling_on_sc=False))
    def k(tbl, idx_hbm, out,
          idx_v, rows0, rows1, accum0, accum1, semg0, semg1, semw0, semw1):
        cid = lax.axis_index("c")
        sid = lax.axis_index("s")
        wid = sid * NC + cid
        pltpu.sync_copy(
            idx_hbm.at[pl.ds(0, T), pl.ds(0, M),
                       pl.ds(wid * BATW, BATW)], idx_v)

        def fire(c, buf, sem):
            for par in range(2):
                for t in range(T):
                    pltpu.async_copy(
                        tbl.at[idx_v.at[t, 2 * c + par]],
                        buf.at[pl.ds((par * T + t) * BATW, BATW)],
                        sem)

        def drain_g(buf, sem):
            pltpu.make_async_copy(tbl.at[pl.ds(0, RR)], buf, sem).wait()

        def pool(buf, acc):
            @pl.loop(0, BATW)
            def _pool(bb):
                for par in range(2):
                    base = par * T * BATW + bb
                    for d in range(DIM // L):
                        sl = pl.ds(d * L, L)
                        v01 = buf[base, sl] + buf[base + BATW, sl]
                        v23 = (buf[base + 2 * BATW, sl]
                               + buf[base + 3 * BATW, sl])
                        v45 = (buf[base + 4 * BATW, sl]
                               + buf[base + 5 * BATW, sl])
                        acc[par, bb, sl] = (v01 + v23) + v45

        def wb(c, acc, semw):
            for par in range(2):
                pltpu.async_copy(
                    acc.at[par],
                    out.at[pl.ds(wid * BATW, BATW), c,
                           pl.ds(par * DIM, DIM)], semw)

        def drain_w(acc, semw):
            for par in range(2):
                pltpu.make_async_copy(
                    out.at[pl.ds(0, BATW), 0, pl.ds(0, DIM)],
                    acc.at[par], semw).wait()

        fire(0, rows0, semg0)

        @pl.loop(0, NCH - 1, step=2)
        def _chunks(c):
            fire(c + 1, rows1, semg1)
            drain_g(rows0, semg0)

            @pl.when(c >= 2)
            def _():
                drain_w(accum0, semw0)

            pool(rows0, accum0)
            wb(c, accum0, semw0)

            @pl.when(c + 2 < NCH)
            def _():
                fire(c + 2, rows0, semg0)

            drain_g(rows1, semg1)

            @pl.when(c >= 2)
            def _():
                drain_w(accum1, semw1)

            pool(rows1, accum1)
            wb(c + 1, accum1, semw1)

        drain_g(rows0, semg0)
        drain_w(accum0, semw0)
        pool(rows0, accum0)
        wb(NCH - 1, accum0, semw0)
        drain_w(accum0, semw0)
        drain_w(accum1, semw1)

        @pl.loop(0, BATW)
        def _z(bb):
            for d in range(DIM // L):
                rows0[bb, pl.ds(d * L, L)] = jnp.zeros((L,), jnp.float32)

        for i in range(JP - NCH):
            for par in range(2):
                pltpu.async_copy(
                    rows0.at[pl.ds(0, BATW)],
                    out.at[pl.ds(wid * BATW, BATW), NCH + i,
                           pl.ds(par * DIM, DIM)], semg0)
        for i in range(2 * (JP - NCH)):
            pltpu.make_async_copy(
                out.at[pl.ds(0, BATW), 0, pl.ds(0, DIM)],
                rows0.at[pl.ds(0, BATW)], semg0).wait()

    return k(tbl_in, idx_arr)


def _tc_hops(s1, s2, s3):
    Bb = 256

    def body(s1_ref, s2_ref, s3_ref, o_ref):
        m1 = lax.broadcasted_iota(jnp.int32, (Bb, JP, 1), 1) < 25
        lane_e = lax.broadcasted_iota(jnp.int32, (Bb, JP, 2 * DIM), 2) < DIM
        s1 = s1_ref[...]
        s2 = s2_ref[...]
        s3 = s3_ref[...]
        su1 = jnp.sum(s1, axis=1, keepdims=True)
        u = (su1[:, :, 0:DIM] + su1[:, :, DIM:2 * DIM]) / float(M)
        neg = jnp.float32(-1e30)
        for sa, sc in ((s1, s2), (s2, s3)):
            u2 = jnp.concatenate([u, u], axis=2)
            prod = sa * u2
            ae = jnp.sum(prod[:, :, 0:DIM], axis=2, keepdims=True)
            ao = jnp.sum(prod[:, :, DIM:2 * DIM], axis=2, keepdims=True)
            ae = jnp.where(m1, ae, neg)
            ao = jnp.where(m1, ao, neg)
            mx = jnp.maximum(jnp.max(ae, axis=1, keepdims=True),
                             jnp.max(ao, axis=1, keepdims=True))
            ee = jnp.exp(ae - mx)
            eo = jnp.exp(ao - mx)
            z = jnp.sum(ee, axis=1, keepdims=True) + jnp.sum(
                eo, axis=1, keepdims=True)
            p = jnp.where(lane_e, ee, eo) / z
            o2 = jnp.sum(sc * p, axis=1, keepdims=True)
            u = u + o2[:, :, 0:DIM] + o2[:, :, DIM:2 * DIM]
        o_ref[...] = u

    out = pl.pallas_call(
        body,
        grid=(B // Bb,),
        in_specs=[
            pl.BlockSpec((Bb, JP, 2 * DIM), lambda i: (i, 0, 0)),
            pl.BlockSpec((Bb, JP, 2 * DIM), lambda i: (i, 0, 0)),
            pl.BlockSpec((Bb, JP, 2 * DIM), lambda i: (i, 0, 0)),
        ],
        out_specs=pl.BlockSpec((Bb, 1, DIM), lambda i: (i, 0, 0)),
        out_shape=jax.ShapeDtypeStruct((B, 1, DIM), jnp.float32),
    )(s1, s2, s3)
    return out.reshape(B, DIM)


def kernel(story, C_0, C_1, C_2, C_3):
    tp12 = jnp.concatenate(
        [C_1.at[PAD].set(0.0), C_2.at[PAD].set(0.0)], axis=1)
    t3 = jnp.pad(C_3, ((0, 0), (0, DIM))).reshape(2 * VOCAB, DIM)
    js = jnp.transpose(story, (2, 0, 1))
    idx3 = jnp.where(js == PAD, 1, js * 2)
    S1, S2 = _sc_pair_gather(tp12, js)
    S3 = _sc_pooled_gather(t3, idx3)
    return _tc_hops(S1, S2, S3)
